# baseline (speedup 1.0000x reference)
import functools

import jax
import jax.numpy as jnp
from jax import lax
from jax.experimental import pallas as pl
from jax.experimental.pallas import tpu as pltpu
from jax.experimental.pallas import tpu_sc as plsc

N = 10000
D = 128
E = 320000
NPAD = 10240
NC = 2
NS = 16
NW = NC * NS
EPW = NPAD
CH = 128
NCH = EPW // CH
EPAD = NW * EPW
RPT = NPAD // NS
RPO = N // NS
GW = 20

def _deg_body(dst_hbm, deg_hbm, dst_v, ones_v, z_v, deg_sh):
    c = lax.axis_index("c")
    s = lax.axis_index("s")
    wid = c * NS + s
    pltpu.sync_copy(dst_hbm.at[wid], dst_v)
    for k in range(CH // 16):
        ones_v[pl.ds(16 * k, 16)] = jnp.ones((16,), jnp.float32)
    for k in range(RPT // 16):
        z_v[pl.ds(16 * k, 16)] = jnp.zeros((16,), jnp.float32)
    pltpu.sync_copy(z_v, deg_sh.at[pl.ds(s * RPT, RPT)])
    plsc.subcore_barrier()

    def body(j, carry):
        pltpu.sync_copy(ones_v, deg_sh.at[dst_v.at[j]], add=True)
        return carry

    lax.fori_loop(0, NCH, body, 0)
    plsc.subcore_barrier()
    pltpu.sync_copy(deg_sh.at[pl.ds(s * RPT, RPT)],
                    deg_hbm.at[c, pl.ds(s * RPT, RPT)])


def _agg_body(g_hbm, ei_hbm, out_hbm, idx_v, buf, out_sh,
              sem_a, sem_b, sem_w0, sem_w1):
    c = lax.axis_index("c")
    s = lax.axis_index("s")
    wid = c * NS + s

    pltpu.async_copy(ei_hbm.at[wid, pl.ds(0, GW)], idx_v.at[0], sem_w0)

    def zrow(r, carry):
        for k in range(D // 16):
            buf[0, r, pl.ds(16 * k, 16)] = jnp.zeros((16,), jnp.float32)
        return carry

    lax.fori_loop(0, CH, zrow, 0)
    for t in range(RPT // CH):
        pltpu.sync_copy(buf.at[0], out_sh.at[pl.ds(s * RPT + t * CH, CH)])
    plsc.subcore_barrier()

    for g in range(NCH // GW):
        p = g & 1
        semw = sem_w0 if p == 0 else sem_w1
        pltpu.make_async_copy(ei_hbm.at[wid, pl.ds(g * GW, GW)], idx_v.at[p],
                              semw).wait()
        if g + 1 < NCH // GW:
            semw2 = sem_w1 if p == 0 else sem_w0
            pltpu.async_copy(ei_hbm.at[wid, pl.ds((g + 1) * GW, GW)],
                             idx_v.at[1 - p], semw2)
        win = idx_v.at[p]
        pltpu.async_copy(g_hbm.at[win.at[0, 0]], buf.at[0], sem_a)

        def body(t, c2, win=win):
            j0 = 2 * t
            pltpu.make_async_copy(g_hbm.at[win.at[j0, 0]], buf.at[0],
                                  sem_a).wait()
            pltpu.async_copy(g_hbm.at[win.at[j0 + 1, 0]], buf.at[1], sem_b)
            pltpu.sync_copy(buf.at[0], out_sh.at[win.at[j0, 1]], add=True)
            pltpu.make_async_copy(g_hbm.at[win.at[j0 + 1, 0]], buf.at[1],
                                  sem_b).wait()

            @pl.when(j0 + 2 < GW)
            def _():
                pltpu.async_copy(g_hbm.at[win.at[j0 + 2, 0]], buf.at[0],
                                 sem_a)

            pltpu.sync_copy(buf.at[1], out_sh.at[win.at[j0 + 1, 1]], add=True)
            return c2

        lax.fori_loop(0, GW // 2, body, 0)

    plsc.subcore_barrier()
    pltpu.sync_copy(out_sh.at[pl.ds(s * RPT, RPT)],
                    out_hbm.at[c, pl.ds(s * RPT, RPT)])


@functools.lru_cache(maxsize=None)
def _sc_kernels():
    mesh = plsc.VectorSubcoreMesh(core_axis_name="c", subcore_axis_name="s",
                                  num_cores=NC, num_subcores=NS)
    deg_k = pl.kernel(
        _deg_body,
        out_type=jax.ShapeDtypeStruct((NC, NPAD), jnp.float32),
        mesh=mesh,
        scratch_types=[
            pltpu.VMEM((NCH, CH), jnp.int32),
            pltpu.VMEM((CH,), jnp.float32),
            pltpu.VMEM((RPT,), jnp.float32),
            pltpu.VMEM_SHARED((NPAD,), jnp.float32),
        ],
    )
    agg_k = pl.kernel(
        _agg_body,
        out_type=jax.ShapeDtypeStruct((NC, NPAD, D), jnp.float32),
        mesh=mesh,
        scratch_types=[
            pltpu.VMEM((2, GW, 2, CH), jnp.int32),
            pltpu.VMEM((2, CH, D), jnp.float32),
            pltpu.VMEM_SHARED((NPAD, D), jnp.float32),
            pltpu.SemaphoreType.DMA,
            pltpu.SemaphoreType.DMA,
            pltpu.SemaphoreType.DMA,
            pltpu.SemaphoreType.DMA,
        ],
    )
    return deg_k, agg_k


_B = 1000


def _mm_body(x_ref, w_ref, h_ref):
    h_ref[...] = jnp.dot(x_ref[...], w_ref[...],
                         preferred_element_type=jnp.float32)


def _mm_tc(x, W1):
    return pl.pallas_call(
        _mm_body,
        grid=(N // _B,),
        in_specs=[
            pl.BlockSpec((_B, D), lambda i: (i, 0)),
            pl.BlockSpec((D, D), lambda i: (0, 0)),
        ],
        out_specs=pl.BlockSpec((_B, D), lambda i: (i, 0)),
        out_shape=jax.ShapeDtypeStruct((N, D), jnp.float32),
    )(x, W1)


def _scale_body(h_ref, d0_ref, d1_ref, g_ref, dinv_ref):
    deg = d0_ref[...] + d1_ref[...] + 1.0
    dinv = lax.rsqrt(deg)
    g_ref[...] = h_ref[...] * dinv
    dinv_ref[...] = dinv


def _scale_tc(h, deg0, deg1):
    return pl.pallas_call(
        _scale_body,
        grid=(N // _B,),
        in_specs=[
            pl.BlockSpec((_B, D), lambda i: (i, 0)),
            pl.BlockSpec((_B, 1), lambda i: (i, 0)),
            pl.BlockSpec((_B, 1), lambda i: (i, 0)),
        ],
        out_specs=[
            pl.BlockSpec((_B, D), lambda i: (i, 0)),
            pl.BlockSpec((_B, 1), lambda i: (i, 0)),
        ],
        out_shape=[
            jax.ShapeDtypeStruct((N, D), jnp.float32),
            jax.ShapeDtypeStruct((N, 1), jnp.float32),
        ],
    )(h, deg0, deg1)


def _mid_body(a0_ref, a1_ref, g1_ref, dinv_ref, b1_ref, w2_ref, g2_ref):
    agg = a0_ref[...] + a1_ref[...] + g1_ref[...]
    z = agg * dinv_ref[...] + b1_ref[...]
    z = jnp.maximum(z, 0.0)
    g2_ref[...] = jnp.dot(z, w2_ref[...],
                          preferred_element_type=jnp.float32) * dinv_ref[...]


def _mid_tc(a0, a1, g1, dinv, b1, W2):
    return pl.pallas_call(
        _mid_body,
        grid=(N // _B,),
        in_specs=[
            pl.BlockSpec((_B, D), lambda i: (i, 0)),
            pl.BlockSpec((_B, D), lambda i: (i, 0)),
            pl.BlockSpec((_B, D), lambda i: (i, 0)),
            pl.BlockSpec((_B, 1), lambda i: (i, 0)),
            pl.BlockSpec((1, D), lambda i: (0, 0)),
            pl.BlockSpec((D, D), lambda i: (0, 0)),
        ],
        out_specs=pl.BlockSpec((_B, D), lambda i: (i, 0)),
        out_shape=jax.ShapeDtypeStruct((N, D), jnp.float32),
    )(a0, a1, g1, dinv, b1, W2)


def _final_body(a0_ref, a1_ref, g2_ref, dinv_ref, b2_ref, o_ref):
    z = (a0_ref[...] + a1_ref[...] + g2_ref[...]) * dinv_ref[...] + b2_ref[...]
    m = jnp.max(z, axis=1, keepdims=True)
    lse = m + jnp.log(jnp.sum(jnp.exp(z - m), axis=1, keepdims=True))
    o_ref[...] = z - lse


def _final_tc(a0, a1, g2, dinv, b2):
    return pl.pallas_call(
        _final_body,
        grid=(N // _B,),
        in_specs=[
            pl.BlockSpec((_B, D), lambda i: (i, 0)),
            pl.BlockSpec((_B, D), lambda i: (i, 0)),
            pl.BlockSpec((_B, D), lambda i: (i, 0)),
            pl.BlockSpec((_B, 1), lambda i: (i, 0)),
            pl.BlockSpec((1, D), lambda i: (0, 0)),
        ],
        out_specs=pl.BlockSpec((_B, D), lambda i: (i, 0)),
        out_shape=jax.ShapeDtypeStruct((N, D), jnp.float32),
    )(a0, a1, g2, dinv, b2)


def kernel(x, edge_index, W1, b1, W2, b2):
    ei = edge_index.astype(jnp.int32)
    pad_src = jnp.arange(EPAD - E, dtype=jnp.int32) % N
    pad_dst = N + jnp.arange(EPAD - E, dtype=jnp.int32) % (NPAD - N)

    def layout(v, pad):
        return jnp.concatenate([v, pad]).reshape(NCH, NW, CH).transpose(1, 0, 2)

    src = layout(ei[0], pad_src)
    dst = layout(ei[1], pad_dst)
    ei_comb = jnp.stack([src, dst], axis=2)

    deg_kernel, agg_kernel = _sc_kernels()
    degp = deg_kernel(dst)
    h1 = _mm_tc(x, W1)
    deg0 = degp[0].reshape(NPAD, 1)
    deg1 = degp[1].reshape(NPAD, 1)
    g1, dinv = _scale_tc(h1, deg0, deg1)

    A1 = agg_kernel(g1, ei_comb)
    g2 = _mid_tc(A1[0], A1[1], g1, dinv, b1.reshape(1, D), W2)

    A2 = agg_kernel(g2, ei_comb)
    return _final_tc(A2[0], A2[1], g2, dinv, b2.reshape(1, D))


# speedup vs baseline: 1.0070x; 1.0070x over previous
import functools

import jax
import jax.numpy as jnp
from jax import lax
from jax.experimental import pallas as pl
from jax.experimental.pallas import tpu as pltpu
from jax.experimental.pallas import tpu_sc as plsc

N = 10000
D = 128
E = 320000
NPAD = 10240
NC = 2
NS = 16
NW = NC * NS
EPW = NPAD
CH = 128
NCH = EPW // CH
EPAD = NW * EPW
RPT = NPAD // NS
RPO = N // NS
GW = 20

def _deg_body(dst_hbm, deg_hbm, dst_v, ones_v, z_v, deg_sh):
    c = lax.axis_index("c")
    s = lax.axis_index("s")
    wid = c * NS + s
    pltpu.sync_copy(dst_hbm.at[wid], dst_v)
    for k in range(CH // 16):
        ones_v[pl.ds(16 * k, 16)] = jnp.ones((16,), jnp.float32)
    for k in range(RPT // 16):
        z_v[pl.ds(16 * k, 16)] = jnp.zeros((16,), jnp.float32)
    pltpu.sync_copy(z_v, deg_sh.at[pl.ds(s * RPT, RPT)])
    plsc.subcore_barrier()

    def body(j, carry):
        pltpu.sync_copy(ones_v, deg_sh.at[dst_v.at[j]], add=True)
        return carry

    lax.fori_loop(0, NCH, body, 0)
    plsc.subcore_barrier()
    pltpu.sync_copy(deg_sh.at[pl.ds(s * RPT, RPT)],
                    deg_hbm.at[c, pl.ds(s * RPT, RPT)])


def _agg_body(g_hbm, ei_hbm, out_hbm, idx_v, buf, out_sh,
              sem_a, sem_b, sem_w0, sem_w1):
    c = lax.axis_index("c")
    s = lax.axis_index("s")
    wid = c * NS + s

    pltpu.async_copy(ei_hbm.at[wid, pl.ds(0, GW)], idx_v.at[0], sem_w0)

    def zrow(r, carry):
        for k in range(D // 16):
            buf[0, r, pl.ds(16 * k, 16)] = jnp.zeros((16,), jnp.float32)
        return carry

    lax.fori_loop(0, CH, zrow, 0)
    for t in range(RPT // CH):
        pltpu.sync_copy(buf.at[0], out_sh.at[pl.ds(s * RPT + t * CH, CH)])
    plsc.subcore_barrier()

    for g in range(NCH // GW):
        p = g & 1
        semw = sem_w0 if p == 0 else sem_w1
        pltpu.make_async_copy(ei_hbm.at[wid, pl.ds(g * GW, GW)], idx_v.at[p],
                              semw).wait()
        if g + 1 < NCH // GW:
            semw2 = sem_w1 if p == 0 else sem_w0
            pltpu.async_copy(ei_hbm.at[wid, pl.ds((g + 1) * GW, GW)],
                             idx_v.at[1 - p], semw2)
        win = idx_v.at[p]
        pltpu.async_copy(g_hbm.at[win.at[0, 0]], buf.at[0], sem_a)

        def body(t, c2, win=win):
            j0 = 2 * t
            pltpu.make_async_copy(g_hbm.at[win.at[j0, 0]], buf.at[0],
                                  sem_a).wait()
            pltpu.async_copy(g_hbm.at[win.at[j0 + 1, 0]], buf.at[1], sem_b)
            pltpu.sync_copy(buf.at[0], out_sh.at[win.at[j0, 1]], add=True)
            pltpu.make_async_copy(g_hbm.at[win.at[j0 + 1, 0]], buf.at[1],
                                  sem_b).wait()

            @pl.when(j0 + 2 < GW)
            def _():
                pltpu.async_copy(g_hbm.at[win.at[j0 + 2, 0]], buf.at[0],
                                 sem_a)

            pltpu.sync_copy(buf.at[1], out_sh.at[win.at[j0 + 1, 1]], add=True)
            return c2

        lax.fori_loop(0, GW // 2, body, 0)

    plsc.subcore_barrier()
    pltpu.sync_copy(out_sh.at[pl.ds(s * RPT, RPT)],
                    out_hbm.at[c, pl.ds(s * RPT, RPT)])


@functools.lru_cache(maxsize=None)
def _sc_kernels():
    mesh = plsc.VectorSubcoreMesh(core_axis_name="c", subcore_axis_name="s",
                                  num_cores=NC, num_subcores=NS)
    deg_k = pl.kernel(
        _deg_body,
        out_type=jax.ShapeDtypeStruct((NC, NPAD), jnp.float32),
        mesh=mesh,
        scratch_types=[
            pltpu.VMEM((NCH, CH), jnp.int32),
            pltpu.VMEM((CH,), jnp.float32),
            pltpu.VMEM((RPT,), jnp.float32),
            pltpu.VMEM_SHARED((NPAD,), jnp.float32),
        ],
    )
    agg_k = pl.kernel(
        _agg_body,
        out_type=jax.ShapeDtypeStruct((NC, NPAD, D), jnp.float32),
        mesh=mesh,
        scratch_types=[
            pltpu.VMEM((2, GW, 2, CH), jnp.int32),
            pltpu.VMEM((2, CH, D), jnp.float32),
            pltpu.VMEM_SHARED((NPAD, D), jnp.float32),
            pltpu.SemaphoreType.DMA,
            pltpu.SemaphoreType.DMA,
            pltpu.SemaphoreType.DMA,
            pltpu.SemaphoreType.DMA,
        ],
    )
    return deg_k, agg_k


_B = 1000


def _prep_body(x_ref, w_ref, d0_ref, d1_ref, g_ref, dinv_ref):
    deg = d0_ref[...] + d1_ref[...] + 1.0
    dinv = lax.rsqrt(deg)
    h = jnp.dot(x_ref[...], w_ref[...], preferred_element_type=jnp.float32)
    g_ref[...] = h * dinv
    dinv_ref[...] = dinv


def _prep_tc(x, W1, deg0, deg1):
    return pl.pallas_call(
        _prep_body,
        grid=(N // _B,),
        in_specs=[
            pl.BlockSpec((_B, D), lambda i: (i, 0)),
            pl.BlockSpec((D, D), lambda i: (0, 0)),
            pl.BlockSpec((_B, 1), lambda i: (i, 0)),
            pl.BlockSpec((_B, 1), lambda i: (i, 0)),
        ],
        out_specs=[
            pl.BlockSpec((_B, D), lambda i: (i, 0)),
            pl.BlockSpec((_B, 1), lambda i: (i, 0)),
        ],
        out_shape=[
            jax.ShapeDtypeStruct((N, D), jnp.float32),
            jax.ShapeDtypeStruct((N, 1), jnp.float32),
        ],
    )(x, W1, deg0, deg1)


def _mid_body(a0_ref, a1_ref, g1_ref, dinv_ref, b1_ref, w2_ref, g2_ref):
    agg = a0_ref[...] + a1_ref[...] + g1_ref[...]
    z = agg * dinv_ref[...] + b1_ref[...]
    z = jnp.maximum(z, 0.0)
    g2_ref[...] = jnp.dot(z, w2_ref[...],
                          preferred_element_type=jnp.float32) * dinv_ref[...]


def _mid_tc(a0, a1, g1, dinv, b1, W2):
    return pl.pallas_call(
        _mid_body,
        grid=(N // _B,),
        in_specs=[
            pl.BlockSpec((_B, D), lambda i: (i, 0)),
            pl.BlockSpec((_B, D), lambda i: (i, 0)),
            pl.BlockSpec((_B, D), lambda i: (i, 0)),
            pl.BlockSpec((_B, 1), lambda i: (i, 0)),
            pl.BlockSpec((1, D), lambda i: (0, 0)),
            pl.BlockSpec((D, D), lambda i: (0, 0)),
        ],
        out_specs=pl.BlockSpec((_B, D), lambda i: (i, 0)),
        out_shape=jax.ShapeDtypeStruct((N, D), jnp.float32),
    )(a0, a1, g1, dinv, b1, W2)


def _final_body(a0_ref, a1_ref, g2_ref, dinv_ref, b2_ref, o_ref):
    z = (a0_ref[...] + a1_ref[...] + g2_ref[...]) * dinv_ref[...] + b2_ref[...]
    m = jnp.max(z, axis=1, keepdims=True)
    lse = m + jnp.log(jnp.sum(jnp.exp(z - m), axis=1, keepdims=True))
    o_ref[...] = z - lse


def _final_tc(a0, a1, g2, dinv, b2):
    return pl.pallas_call(
        _final_body,
        grid=(N // _B,),
        in_specs=[
            pl.BlockSpec((_B, D), lambda i: (i, 0)),
            pl.BlockSpec((_B, D), lambda i: (i, 0)),
            pl.BlockSpec((_B, D), lambda i: (i, 0)),
            pl.BlockSpec((_B, 1), lambda i: (i, 0)),
            pl.BlockSpec((1, D), lambda i: (0, 0)),
        ],
        out_specs=pl.BlockSpec((_B, D), lambda i: (i, 0)),
        out_shape=jax.ShapeDtypeStruct((N, D), jnp.float32),
    )(a0, a1, g2, dinv, b2)


def kernel(x, edge_index, W1, b1, W2, b2):
    ei = edge_index.astype(jnp.int32)
    pad_src = jnp.arange(EPAD - E, dtype=jnp.int32) % N
    pad_dst = N + jnp.arange(EPAD - E, dtype=jnp.int32) % (NPAD - N)

    def layout(v, pad):
        return jnp.concatenate([v, pad]).reshape(NCH, NW, CH).transpose(1, 0, 2)

    src = layout(ei[0], pad_src)
    dst = layout(ei[1], pad_dst)
    ei_comb = jnp.stack([src, dst], axis=2)

    deg_kernel, agg_kernel = _sc_kernels()
    degp = deg_kernel(dst)
    deg0 = degp[0].reshape(NPAD, 1)
    deg1 = degp[1].reshape(NPAD, 1)
    g1, dinv = _prep_tc(x, W1, deg0, deg1)

    A1 = agg_kernel(g1, ei_comb)
    g2 = _mid_tc(A1[0], A1[1], g1, dinv, b1.reshape(1, D), W2)

    A2 = agg_kernel(g2, ei_comb)
    return _final_tc(A2[0], A2[1], g2, dinv, b2.reshape(1, D))


# speedup vs baseline: 1.0103x; 1.0033x over previous
import functools

import jax
import jax.numpy as jnp
from jax import lax
from jax.experimental import pallas as pl
from jax.experimental.pallas import tpu as pltpu
from jax.experimental.pallas import tpu_sc as plsc

N = 10000
D = 128
E = 320000
NPAD = 10240
NC = 2
NS = 16
NW = NC * NS
EPW = NPAD
CH = 128
NCH = EPW // CH
EPAD = NW * EPW
RPT = NPAD // NS
GW = 20

def _deg_body(dst_hbm, deg_hbm, dst_v, ones_v, z_v, deg_sh):
    c = lax.axis_index("c")
    s = lax.axis_index("s")
    wid = c * NS + s
    pltpu.sync_copy(dst_hbm.at[wid], dst_v)
    for k in range(CH // 16):
        ones_v[pl.ds(16 * k, 16)] = jnp.ones((16,), jnp.float32)
    for k in range(RPT // 16):
        z_v[pl.ds(16 * k, 16)] = jnp.zeros((16,), jnp.float32)
    pltpu.sync_copy(z_v, deg_sh.at[pl.ds(s * RPT, RPT)])
    plsc.subcore_barrier()

    def body(j, carry):
        pltpu.sync_copy(ones_v, deg_sh.at[dst_v.at[j]], add=True)
        return carry

    lax.fori_loop(0, NCH, body, 0)
    plsc.subcore_barrier()
    pltpu.sync_copy(deg_sh.at[pl.ds(s * RPT, RPT)],
                    deg_hbm.at[c, pl.ds(s * RPT, RPT)])


def _agg_body(g_hbm, ei_hbm, out_hbm, idx_v, buf, out_sh,
              sem_a, sem_b, sem_w0, sem_w1):
    c = lax.axis_index("c")
    s = lax.axis_index("s")
    wid = c * NS + s

    pltpu.async_copy(ei_hbm.at[wid, pl.ds(0, GW)], idx_v.at[0], sem_w0)

    def zrow(r, carry):
        for k in range(D // 16):
            buf[0, r, pl.ds(16 * k, 16)] = jnp.zeros((16,), jnp.float32)
        return carry

    lax.fori_loop(0, CH, zrow, 0)
    for t in range(RPT // CH):
        pltpu.sync_copy(buf.at[0], out_sh.at[pl.ds(s * RPT + t * CH, CH)])
    plsc.subcore_barrier()

    for g in range(NCH // GW):
        p = g & 1
        semw = sem_w0 if p == 0 else sem_w1
        pltpu.make_async_copy(ei_hbm.at[wid, pl.ds(g * GW, GW)], idx_v.at[p],
                              semw).wait()
        if g + 1 < NCH // GW:
            semw2 = sem_w1 if p == 0 else sem_w0
            pltpu.async_copy(ei_hbm.at[wid, pl.ds((g + 1) * GW, GW)],
                             idx_v.at[1 - p], semw2)
        win = idx_v.at[p]
        pltpu.async_copy(g_hbm.at[win.at[0, 0]], buf.at[0], sem_a)

        def body(t, c2, win=win):
            j0 = 2 * t
            pltpu.make_async_copy(g_hbm.at[win.at[j0, 0]], buf.at[0],
                                  sem_a).wait()
            pltpu.async_copy(g_hbm.at[win.at[j0 + 1, 0]], buf.at[1], sem_b)
            pltpu.sync_copy(buf.at[0], out_sh.at[win.at[j0, 1]], add=True)
            pltpu.make_async_copy(g_hbm.at[win.at[j0 + 1, 0]], buf.at[1],
                                  sem_b).wait()

            @pl.when(j0 + 2 < GW)
            def _():
                pltpu.async_copy(g_hbm.at[win.at[j0 + 2, 0]], buf.at[0],
                                 sem_a)

            pltpu.sync_copy(buf.at[1], out_sh.at[win.at[j0 + 1, 1]], add=True)
            return c2

        lax.fori_loop(0, GW // 2, body, 0)

    plsc.subcore_barrier()
    pltpu.sync_copy(out_sh.at[pl.ds(s * RPT, RPT)],
                    out_hbm.at[c, pl.ds(s * RPT, RPT)])


@functools.lru_cache(maxsize=None)
def _sc_kernels():
    mesh = plsc.VectorSubcoreMesh(core_axis_name="c", subcore_axis_name="s",
                                  num_cores=NC, num_subcores=NS)
    deg_k = pl.kernel(
        _deg_body,
        out_type=jax.ShapeDtypeStruct((NC, NPAD), jnp.float32),
        mesh=mesh,
        scratch_types=[
            pltpu.VMEM((NCH, CH), jnp.int32),
            pltpu.VMEM((CH,), jnp.float32),
            pltpu.VMEM((RPT,), jnp.float32),
            pltpu.VMEM_SHARED((NPAD,), jnp.float32),
        ],
    )
    agg_k = pl.kernel(
        _agg_body,
        out_type=jax.ShapeDtypeStruct((NC, NPAD, D), jnp.float32),
        mesh=mesh,
        scratch_types=[
            pltpu.VMEM((2, GW, 2, CH), jnp.int32),
            pltpu.VMEM((2, CH, D), jnp.float32),
            pltpu.VMEM_SHARED((NPAD, D), jnp.float32),
            pltpu.SemaphoreType.DMA,
            pltpu.SemaphoreType.DMA,
            pltpu.SemaphoreType.DMA,
            pltpu.SemaphoreType.DMA,
        ],
    )
    return deg_k, agg_k


_B = 1000


def _prep_body(x_ref, w_ref, d0_ref, d1_ref, g_ref, dinv_ref):
    deg = d0_ref[...] + d1_ref[...] + 1.0
    dinv = lax.rsqrt(deg)
    h = jnp.dot(x_ref[...], w_ref[...], preferred_element_type=jnp.float32)
    g_ref[...] = h * dinv
    dinv_ref[...] = dinv


def _prep_tc(x, W1, deg0, deg1):
    return pl.pallas_call(
        _prep_body,
        grid=(N // _B,),
        in_specs=[
            pl.BlockSpec((_B, D), lambda i: (i, 0)),
            pl.BlockSpec((D, D), lambda i: (0, 0)),
            pl.BlockSpec((_B, 1), lambda i: (i, 0)),
            pl.BlockSpec((_B, 1), lambda i: (i, 0)),
        ],
        out_specs=[
            pl.BlockSpec((_B, D), lambda i: (i, 0)),
            pl.BlockSpec((_B, 1), lambda i: (i, 0)),
        ],
        out_shape=[
            jax.ShapeDtypeStruct((N, D), jnp.float32),
            jax.ShapeDtypeStruct((N, 1), jnp.float32),
        ],
    )(x, W1, deg0, deg1)


def _mid_body(a0_ref, a1_ref, g1_ref, dinv_ref, b1_ref, w2_ref, g2_ref):
    agg = a0_ref[...] + a1_ref[...] + g1_ref[...]
    z = agg * dinv_ref[...] + b1_ref[...]
    z = jnp.maximum(z, 0.0)
    g2_ref[...] = jnp.dot(z, w2_ref[...],
                          preferred_element_type=jnp.float32) * dinv_ref[...]


def _mid_tc(a0, a1, g1, dinv, b1, W2):
    return pl.pallas_call(
        _mid_body,
        grid=(N // _B,),
        in_specs=[
            pl.BlockSpec((_B, D), lambda i: (i, 0)),
            pl.BlockSpec((_B, D), lambda i: (i, 0)),
            pl.BlockSpec((_B, D), lambda i: (i, 0)),
            pl.BlockSpec((_B, 1), lambda i: (i, 0)),
            pl.BlockSpec((1, D), lambda i: (0, 0)),
            pl.BlockSpec((D, D), lambda i: (0, 0)),
        ],
        out_specs=pl.BlockSpec((_B, D), lambda i: (i, 0)),
        out_shape=jax.ShapeDtypeStruct((N, D), jnp.float32),
    )(a0, a1, g1, dinv, b1, W2)


def _final_body(a0_ref, a1_ref, g2_ref, dinv_ref, b2_ref, o_ref):
    z = (a0_ref[...] + a1_ref[...] + g2_ref[...]) * dinv_ref[...] + b2_ref[...]
    m = jnp.max(z, axis=1, keepdims=True)
    lse = m + jnp.log(jnp.sum(jnp.exp(z - m), axis=1, keepdims=True))
    o_ref[...] = z - lse


def _final_tc(a0, a1, g2, dinv, b2):
    return pl.pallas_call(
        _final_body,
        grid=(N // _B,),
        in_specs=[
            pl.BlockSpec((_B, D), lambda i: (i, 0)),
            pl.BlockSpec((_B, D), lambda i: (i, 0)),
            pl.BlockSpec((_B, D), lambda i: (i, 0)),
            pl.BlockSpec((_B, 1), lambda i: (i, 0)),
            pl.BlockSpec((1, D), lambda i: (0, 0)),
        ],
        out_specs=pl.BlockSpec((_B, D), lambda i: (i, 0)),
        out_shape=jax.ShapeDtypeStruct((N, D), jnp.float32),
    )(a0, a1, g2, dinv, b2)


def kernel(x, edge_index, W1, b1, W2, b2):
    ei = edge_index.astype(jnp.int32)
    pad_src = jnp.arange(EPAD - E, dtype=jnp.int32) % N
    pad_dst = N + jnp.arange(EPAD - E, dtype=jnp.int32) % (NPAD - N)

    def layout(v, pad):
        return jnp.concatenate([v, pad]).reshape(NCH, NW, CH).transpose(1, 0, 2)

    src = layout(ei[0], pad_src)
    dst = layout(ei[1], pad_dst)
    ei_comb = jnp.stack([src, dst], axis=2)

    deg_kernel, agg_kernel = _sc_kernels()
    degp = deg_kernel(dst)
    deg0 = degp[0].reshape(NPAD, 1)
    deg1 = degp[1].reshape(NPAD, 1)
    g1, dinv = _prep_tc(x, W1, deg0, deg1)

    A1 = agg_kernel(g1, ei_comb)
    g2 = _mid_tc(A1[0], A1[1], g1, dinv, b1.reshape(1, D), W2)

    A2 = agg_kernel(g2, ei_comb)
    return _final_tc(A2[0], A2[1], g2, dinv, b2.reshape(1, D))
